# baseline (speedup 1.0000x reference)
import functools

import jax
import jax.numpy as jnp
from jax import lax
from jax.experimental import pallas as pl
from jax.experimental.pallas import tpu as pltpu
from jax.experimental.pallas import tpu_sc as plsc

BATCH = 16384
EMBED = 32
IDX_CHUNK = 128


def _make_gather(num_cores: int, num_workers: int):
    b_per_w = BATCH // num_workers
    n_chunks = b_per_w // IDX_CHUNK
    mesh = plsc.VectorSubcoreMesh(core_axis_name="c", subcore_axis_name="s")

    @functools.partial(
        pl.kernel,
        mesh=mesh,
        compiler_params=pltpu.CompilerParams(use_tc_tiling_on_sc=False),
        out_type=(
            jax.ShapeDtypeStruct((BATCH, EMBED), jnp.float32),
            jax.ShapeDtypeStruct((BATCH, EMBED), jnp.float32),
        ),
        scratch_types=[
            pltpu.VMEM((n_chunks, IDX_CHUNK), jnp.int32),
            pltpu.VMEM((n_chunks, IDX_CHUNK), jnp.int32),
            pltpu.VMEM((b_per_w, EMBED), jnp.float32),
            pltpu.VMEM((b_per_w, EMBED), jnp.float32),
            pltpu.SemaphoreType.DMA,
        ],
    )
    def gather_kernel(uid_hbm, mid_hbm, utab_hbm, mtab_hbm, uout_hbm, mout_hbm,
                      uidx_v, midx_v, urows_v, mrows_v, sem):
        wid = lax.axis_index("s") * num_cores + lax.axis_index("c")
        base = wid * b_per_w
        pltpu.sync_copy(uid_hbm.at[wid], uidx_v)
        pltpu.sync_copy(mid_hbm.at[wid], midx_v)
        copies = []
        for j in range(n_chunks):
            copies.append(pltpu.async_copy(
                utab_hbm.at[uidx_v.at[j]],
                urows_v.at[pl.ds(j * IDX_CHUNK, IDX_CHUNK)], sem))
            copies.append(pltpu.async_copy(
                mtab_hbm.at[midx_v.at[j]],
                mrows_v.at[pl.ds(j * IDX_CHUNK, IDX_CHUNK)], sem))
        for c in copies:
            c.wait()
        pltpu.sync_copy(urows_v, uout_hbm.at[pl.ds(base, b_per_w)])
        pltpu.sync_copy(mrows_v, mout_hbm.at[pl.ds(base, b_per_w)])

    return gather_kernel, n_chunks


def _mlp_body(u_ref, m_ref, w1u_ref, w1m_ref, b1_ref, w2_ref, b2_ref,
              w3_ref, b3_ref, o_ref):
    h1 = jnp.dot(u_ref[...], w1u_ref[...], preferred_element_type=jnp.float32)
    h1 += jnp.dot(m_ref[...], w1m_ref[...], preferred_element_type=jnp.float32)
    h1 = jnp.maximum(h1 + b1_ref[...], 0.0)
    h2 = jnp.dot(h1, w2_ref[...], preferred_element_type=jnp.float32)
    h2 = jnp.maximum(h2 + b2_ref[...], 0.0)
    o_ref[...] = jnp.sum(h2 * w3_ref[...], axis=1, keepdims=True) + b3_ref[...]


def _mlp_call(u, m, W1u, W1m, b1, W2, b2, W3r, b3, blk: int):
    grid = (BATCH // blk,)
    return pl.pallas_call(
        _mlp_body,
        grid=grid,
        in_specs=[
            pl.BlockSpec((blk, EMBED), lambda i: (i, 0)),
            pl.BlockSpec((blk, EMBED), lambda i: (i, 0)),
            pl.BlockSpec((EMBED, 256), lambda i: (0, 0)),
            pl.BlockSpec((EMBED, 256), lambda i: (0, 0)),
            pl.BlockSpec((1, 256), lambda i: (0, 0)),
            pl.BlockSpec((256, 64), lambda i: (0, 0)),
            pl.BlockSpec((1, 64), lambda i: (0, 0)),
            pl.BlockSpec((1, 64), lambda i: (0, 0)),
            pl.BlockSpec((1, 1), lambda i: (0, 0)),
        ],
        out_specs=pl.BlockSpec((blk, 1), lambda i: (i, 0)),
        out_shape=jax.ShapeDtypeStruct((BATCH, 1), jnp.float32),
    )(u, m, W1u, W1m, b1, W2, b2, W3r, b3)


def kernel(userId, movieId, user_table, movie_table, W1, b1, W2, b2, W3, b3):
    info = plsc.get_sparse_core_info()
    num_workers = info.num_cores * info.num_subcores
    gather_kernel, n_chunks = _make_gather(info.num_cores, num_workers)

    uid = userId.astype(jnp.int32).reshape(num_workers, n_chunks, IDX_CHUNK)
    mid = movieId.astype(jnp.int32).reshape(num_workers, n_chunks, IDX_CHUNK)
    u_emb, m_emb = gather_kernel(uid, mid, user_table, movie_table)

    W1u = W1[:EMBED]
    W1m = W1[EMBED:]
    return _mlp_call(u_emb, m_emb, W1u, W1m, b1.reshape(1, 256), W2,
                     b2.reshape(1, 64), W3.reshape(1, 64), b3.reshape(1, 1),
                     blk=1024)


# speedup vs baseline: 2.1797x; 2.1797x over previous
import functools

import jax
import jax.numpy as jnp
from jax import lax
from jax.experimental import pallas as pl
from jax.experimental.pallas import tpu as pltpu
from jax.experimental.pallas import tpu_sc as plsc

BATCH = 16384
EMBED = 32
ROW = 128
VOCAB = 1000000
IDX_CHUNK = 128


def _make_gather(num_cores: int, num_workers: int):
    b_per_w = BATCH // num_workers
    n_chunks = b_per_w // IDX_CHUNK
    mesh = plsc.VectorSubcoreMesh(core_axis_name="c", subcore_axis_name="s")

    @functools.partial(
        pl.kernel,
        mesh=mesh,
        compiler_params=pltpu.CompilerParams(use_tc_tiling_on_sc=False),
        out_type=(
            jax.ShapeDtypeStruct((BATCH, ROW), jnp.float32),
            jax.ShapeDtypeStruct((BATCH, ROW), jnp.float32),
        ),
        scratch_types=[
            pltpu.VMEM((n_chunks, IDX_CHUNK), jnp.int32),
            pltpu.VMEM((b_per_w, ROW), jnp.float32),
            pltpu.SemaphoreType.DMA,
        ],
    )
    def gather_kernel(uid_hbm, mid_hbm, utab_hbm, mtab_hbm, uout_hbm, mout_hbm,
                      idx_v, rows_v, sem):
        wid = lax.axis_index("s") * num_cores + lax.axis_index("c")
        base = wid * b_per_w
        for tab_hbm, id_hbm, out_hbm in ((utab_hbm, uid_hbm, uout_hbm),
                                         (mtab_hbm, mid_hbm, mout_hbm)):
            pltpu.sync_copy(id_hbm.at[wid], idx_v)
            copies = []
            for j in range(n_chunks):
                copies.append(pltpu.async_copy(
                    tab_hbm.at[idx_v.at[j]],
                    rows_v.at[pl.ds(j * IDX_CHUNK, IDX_CHUNK)], sem))
            for c in copies:
                c.wait()
            pltpu.sync_copy(rows_v, out_hbm.at[pl.ds(base, b_per_w)])

    return gather_kernel, n_chunks


REPACK_BLK = 2048
N_BLOCKS = -(-VOCAB // (4 * REPACK_BLK))
VQ_PAD = N_BLOCKS * REPACK_BLK


def _repack_body(t_ref, eye_ref, o_ref):
    x = t_ref[...]
    n = o_ref.shape[0]
    x4 = jnp.concatenate([x[:, n * a:n * (a + 1)] for a in range(4)],
                         axis=0)
    o_ref[...] = jax.lax.dot_general(
        x4, eye_ref[...], (((0,), (0,)), ((), ())),
        preferred_element_type=jnp.float32,
        precision=jax.lax.Precision.HIGHEST)


def _repack(table_t, eye):
    return pl.pallas_call(
        _repack_body,
        grid=(N_BLOCKS,),
        in_specs=[
            pl.BlockSpec((EMBED, 4 * REPACK_BLK), lambda i: (0, i)),
            pl.BlockSpec((ROW, ROW), lambda i: (0, 0)),
        ],
        out_specs=pl.BlockSpec((REPACK_BLK, ROW), lambda i: (i, 0)),
        out_shape=jax.ShapeDtypeStruct((VQ_PAD, ROW), jnp.float32),
    )(table_t, eye)


def _select_window(rows, rem):
    x = jnp.where(rem == 0, rows[:, 0:32], rows[:, 32:64])
    y = jnp.where(rem == 2, rows[:, 64:96], rows[:, 96:128])
    return jnp.where(rem < 2, x, y)


def _mlp_body(u_ref, m_ref, ur_ref, mr_ref, w1u_ref, w1m_ref, b1_ref, w2_ref,
              b2_ref, w3_ref, b3_ref, o_ref):
    xu = _select_window(u_ref[...], ur_ref[...])
    xm = _select_window(m_ref[...], mr_ref[...])
    h1 = jnp.dot(xu, w1u_ref[...], preferred_element_type=jnp.float32)
    h1 += jnp.dot(xm, w1m_ref[...], preferred_element_type=jnp.float32)
    h1 = jnp.maximum(h1 + b1_ref[...], 0.0)
    h2 = jnp.dot(h1, w2_ref[...], preferred_element_type=jnp.float32)
    h2 = jnp.maximum(h2 + b2_ref[...], 0.0)
    o_ref[...] = jnp.sum(h2 * w3_ref[...], axis=1, keepdims=True) + b3_ref[...]


def _mlp_call(u, m, ur, mr, W1u, W1m, b1, W2, b2, W3r, b3, blk: int):
    grid = (BATCH // blk,)
    return pl.pallas_call(
        _mlp_body,
        grid=grid,
        in_specs=[
            pl.BlockSpec((blk, ROW), lambda i: (i, 0)),
            pl.BlockSpec((blk, ROW), lambda i: (i, 0)),
            pl.BlockSpec((blk, 1), lambda i: (i, 0)),
            pl.BlockSpec((blk, 1), lambda i: (i, 0)),
            pl.BlockSpec((EMBED, 256), lambda i: (0, 0)),
            pl.BlockSpec((EMBED, 256), lambda i: (0, 0)),
            pl.BlockSpec((1, 256), lambda i: (0, 0)),
            pl.BlockSpec((256, 64), lambda i: (0, 0)),
            pl.BlockSpec((1, 64), lambda i: (0, 0)),
            pl.BlockSpec((1, 64), lambda i: (0, 0)),
            pl.BlockSpec((1, 1), lambda i: (0, 0)),
        ],
        out_specs=pl.BlockSpec((blk, 1), lambda i: (i, 0)),
        out_shape=jax.ShapeDtypeStruct((BATCH, 1), jnp.float32),
    )(u, m, ur, mr, W1u, W1m, b1, W2, b2, W3r, b3)


def kernel(userId, movieId, user_table, movie_table, W1, b1, W2, b2, W3, b3):
    info = plsc.get_sparse_core_info()
    num_workers = info.num_cores * info.num_subcores
    gather_kernel, n_chunks = _make_gather(info.num_cores, num_workers)

    uid = userId.astype(jnp.int32)
    mid = movieId.astype(jnp.int32)
    span = 4 * REPACK_BLK
    uq = (REPACK_BLK * (uid // span) + (uid % span) % REPACK_BLK)
    mq = (REPACK_BLK * (mid // span) + (mid % span) % REPACK_BLK)
    uq = uq.reshape(num_workers, n_chunks, IDX_CHUNK)
    mq = mq.reshape(num_workers, n_chunks, IDX_CHUNK)
    eye = jnp.eye(ROW, dtype=jnp.float32)
    tab_u = _repack(user_table.T, eye)
    tab_m = _repack(movie_table.T, eye)
    u128, m128 = gather_kernel(uq, mq, tab_u, tab_m)

    ur = ((uid % span) // REPACK_BLK).reshape(BATCH, 1)
    mr = ((mid % span) // REPACK_BLK).reshape(BATCH, 1)
    return _mlp_call(u128, m128, ur, mr, W1[:EMBED], W1[EMBED:],
                     b1.reshape(1, 256), W2, b2.reshape(1, 64),
                     W3.reshape(1, 64), b3.reshape(1, 1), blk=1024)


# speedup vs baseline: 2.6456x; 1.2137x over previous
import functools

import jax
import jax.numpy as jnp
from jax import lax
from jax.experimental import pallas as pl
from jax.experimental.pallas import tpu as pltpu
from jax.experimental.pallas import tpu_sc as plsc

BATCH = 16384
EMBED = 32
ROW = 128
VOCAB = 1000000
IDX_CHUNK = 128


def _make_gather(num_cores: int, num_workers: int):
    b_per_w = BATCH // num_workers
    n_chunks = b_per_w // IDX_CHUNK
    mesh = plsc.VectorSubcoreMesh(core_axis_name="c", subcore_axis_name="s")

    @functools.partial(
        pl.kernel,
        mesh=mesh,
        compiler_params=pltpu.CompilerParams(use_tc_tiling_on_sc=False),
        out_type=(
            jax.ShapeDtypeStruct((BATCH, ROW), jnp.float32),
            jax.ShapeDtypeStruct((BATCH, ROW), jnp.float32),
        ),
        scratch_types=[
            pltpu.VMEM((n_chunks, IDX_CHUNK), jnp.int32),
            pltpu.VMEM((b_per_w, ROW), jnp.float32),
            pltpu.SemaphoreType.DMA,
        ],
    )
    def gather_kernel(uid_hbm, mid_hbm, utab_hbm, mtab_hbm, uout_hbm, mout_hbm,
                      idx_v, rows_v, sem):
        wid = lax.axis_index("s") * num_cores + lax.axis_index("c")
        base = wid * b_per_w
        for tab_hbm, id_hbm, out_hbm in ((utab_hbm, uid_hbm, uout_hbm),
                                         (mtab_hbm, mid_hbm, mout_hbm)):
            pltpu.sync_copy(id_hbm.at[wid], idx_v)
            copies = []
            for j in range(n_chunks):
                copies.append(pltpu.async_copy(
                    tab_hbm.at[idx_v.at[j]],
                    rows_v.at[pl.ds(j * IDX_CHUNK, IDX_CHUNK)], sem))
            for c in copies:
                c.wait()
            pltpu.sync_copy(rows_v, out_hbm.at[pl.ds(base, b_per_w)])

    return gather_kernel, n_chunks


REPACK_BLK = 2048
N_BLOCKS = -(-VOCAB // (4 * REPACK_BLK))
VQ_PAD = N_BLOCKS * REPACK_BLK


def _repack_body(t_ref, eye_ref, o_ref):
    x = t_ref[...]
    n = o_ref.shape[0]
    x4 = jnp.concatenate([x[:, n * a:n * (a + 1)] for a in range(4)],
                         axis=0)
    o_ref[...] = jax.lax.dot_general(
        x4, eye_ref[...], (((0,), (0,)), ((), ())),
        preferred_element_type=jnp.float32,
        precision=jax.lax.Precision.DEFAULT)


def _repack(table_t, eye):
    return pl.pallas_call(
        _repack_body,
        grid=(N_BLOCKS,),
        in_specs=[
            pl.BlockSpec((EMBED, 4 * REPACK_BLK), lambda i: (0, i)),
            pl.BlockSpec((ROW, ROW), lambda i: (0, 0)),
        ],
        out_specs=pl.BlockSpec((REPACK_BLK, ROW), lambda i: (i, 0)),
        out_shape=jax.ShapeDtypeStruct((VQ_PAD, ROW), jnp.float32),
    )(table_t, eye)


def _select_window(rows, rem):
    x = jnp.where(rem == 0, rows[:, 0:32], rows[:, 32:64])
    y = jnp.where(rem == 2, rows[:, 64:96], rows[:, 96:128])
    return jnp.where(rem < 2, x, y)


def _mlp_body(u_ref, m_ref, ur_ref, mr_ref, w1u_ref, w1m_ref, b1_ref, w2_ref,
              b2_ref, w3_ref, b3_ref, o_ref):
    xu = _select_window(u_ref[...], ur_ref[...])
    xm = _select_window(m_ref[...], mr_ref[...])
    h1 = jnp.dot(xu, w1u_ref[...], preferred_element_type=jnp.float32)
    h1 += jnp.dot(xm, w1m_ref[...], preferred_element_type=jnp.float32)
    h1 = jnp.maximum(h1 + b1_ref[...], 0.0)
    h2 = jnp.dot(h1, w2_ref[...], preferred_element_type=jnp.float32)
    h2 = jnp.maximum(h2 + b2_ref[...], 0.0)
    o_ref[...] = jnp.sum(h2 * w3_ref[...], axis=1, keepdims=True) + b3_ref[...]


def _mlp_call(u, m, ur, mr, W1u, W1m, b1, W2, b2, W3r, b3, blk: int):
    grid = (BATCH // blk,)
    return pl.pallas_call(
        _mlp_body,
        grid=grid,
        in_specs=[
            pl.BlockSpec((blk, ROW), lambda i: (i, 0)),
            pl.BlockSpec((blk, ROW), lambda i: (i, 0)),
            pl.BlockSpec((blk, 1), lambda i: (i, 0)),
            pl.BlockSpec((blk, 1), lambda i: (i, 0)),
            pl.BlockSpec((EMBED, 256), lambda i: (0, 0)),
            pl.BlockSpec((EMBED, 256), lambda i: (0, 0)),
            pl.BlockSpec((1, 256), lambda i: (0, 0)),
            pl.BlockSpec((256, 64), lambda i: (0, 0)),
            pl.BlockSpec((1, 64), lambda i: (0, 0)),
            pl.BlockSpec((1, 64), lambda i: (0, 0)),
            pl.BlockSpec((1, 1), lambda i: (0, 0)),
        ],
        out_specs=pl.BlockSpec((blk, 1), lambda i: (i, 0)),
        out_shape=jax.ShapeDtypeStruct((BATCH, 1), jnp.float32),
    )(u, m, ur, mr, W1u, W1m, b1, W2, b2, W3r, b3)


def kernel(userId, movieId, user_table, movie_table, W1, b1, W2, b2, W3, b3):
    info = plsc.get_sparse_core_info()
    num_workers = info.num_cores * info.num_subcores
    gather_kernel, n_chunks = _make_gather(info.num_cores, num_workers)

    uid = userId.astype(jnp.int32)
    mid = movieId.astype(jnp.int32)
    span = 4 * REPACK_BLK
    uq = (REPACK_BLK * (uid // span) + (uid % span) % REPACK_BLK)
    mq = (REPACK_BLK * (mid // span) + (mid % span) % REPACK_BLK)
    uq = uq.reshape(num_workers, n_chunks, IDX_CHUNK)
    mq = mq.reshape(num_workers, n_chunks, IDX_CHUNK)
    eye = jnp.eye(ROW, dtype=jnp.float32)
    tab_u = _repack(user_table.T, eye)
    tab_m = _repack(movie_table.T, eye)
    u128, m128 = gather_kernel(uq, mq, tab_u, tab_m)

    ur = ((uid % span) // REPACK_BLK).reshape(BATCH, 1)
    mr = ((mid % span) // REPACK_BLK).reshape(BATCH, 1)
    return _mlp_call(u128, m128, ur, mr, W1[:EMBED], W1[EMBED:],
                     b1.reshape(1, 256), W2, b2.reshape(1, 64),
                     W3.reshape(1, 64), b3.reshape(1, 1), blk=1024)
